# Initial kernel scaffold; baseline (speedup 1.0000x reference)
import jax
import jax.numpy as jnp
from jax.experimental import pallas as pl


def kernel(x, edge_index, batch, W1, b1, g1, be1, W2, b2, g2, be2, Wl1, bl1, g3, be3, Wl2, bl2):
    raise NotImplementedError("write your pallas kernel here")


# speedup vs baseline: 6.8084x; 6.8084x over previous
import functools

import jax
import jax.numpy as jnp
from jax import lax
from jax.experimental import pallas as pl
from jax.experimental.pallas import tpu as pltpu
from jax.experimental.pallas import tpu_sc as plsc

N = 10000
NP = 10240
E = 320000
G = 128
D = 128
HD = 64
EPS = 1e-5
NTILES = 32
CH = 128
CPT = 80
EP = NTILES * CPT * CH
DUMMY = NP - 1
RPS = NP // 16
RPT = NP // NTILES

_mesh = plsc.VectorSubcoreMesh(core_axis_name="c", subcore_axis_name="s")
_sc_params = pltpu.CompilerParams(use_tc_tiling_on_sc=False)


@functools.partial(
    pl.kernel,
    mesh=_mesh,
    out_type=jax.ShapeDtypeStruct((2, NP), jnp.float32),
    scratch_types=[
        pltpu.VMEM((CPT, CH), jnp.int32),
        pltpu.VMEM((CH,), jnp.float32),
        pltpu.VMEM((RPS,), jnp.float32),
        pltpu.VMEM_SHARED((NP,), jnp.float32),
    ],
)
def _sc_degree(dst_hbm, out_hbm, dst_v, ones_v, stage_v, acc_sh):
    cid = lax.axis_index("c")
    sid = lax.axis_index("s")
    wid = sid * 2 + cid
    pltpu.sync_copy(dst_hbm.at[pl.ds(wid * CPT, CPT)], dst_v)
    ones16 = jnp.ones((16,), jnp.float32)
    zero16 = jnp.zeros((16,), jnp.float32)

    def _fill(i, _):
        ones_v[pl.ds(i * 16, 16)] = ones16
        return 0

    lax.fori_loop(0, CH // 16, _fill, 0)

    def _fillz(i, _):
        stage_v[pl.ds(i * 16, 16)] = zero16
        return 0

    lax.fori_loop(0, RPS // 16, _fillz, 0)
    pltpu.sync_copy(stage_v, acc_sh.at[pl.ds(sid * RPS, RPS)])
    plsc.subcore_barrier()

    def _scat(j, _):
        pltpu.sync_copy(ones_v, acc_sh.at[dst_v.at[j]], add=True)
        return 0

    lax.fori_loop(0, CPT, _scat, 0)
    plsc.subcore_barrier()
    pltpu.sync_copy(acc_sh.at[pl.ds(sid * RPS, RPS)], stage_v)
    pltpu.sync_copy(stage_v, out_hbm.at[cid, pl.ds(sid * RPS, RPS)])


@functools.partial(
    pl.kernel,
    mesh=_mesh,
    out_type=jax.ShapeDtypeStruct((2, NP, HD), jnp.float32),
    compiler_params=_sc_params,
    scratch_types=[
        pltpu.VMEM((CPT, CH), jnp.int32),
        pltpu.VMEM((CPT, CH), jnp.int32),
        pltpu.VMEM((CH, HD), jnp.float32),
        pltpu.VMEM((128, HD), jnp.float32),
        pltpu.VMEM_SHARED((NP, HD), jnp.float32),
    ],
)
def _sc_mp(hs_hbm, src_hbm, dst_hbm, out_hbm, src_v, dst_v, rows_v, zb, acc_sh):
    cid = lax.axis_index("c")
    sid = lax.axis_index("s")
    wid = sid * 2 + cid
    pltpu.sync_copy(src_hbm.at[pl.ds(wid * CPT, CPT)], src_v)
    pltpu.sync_copy(dst_hbm.at[pl.ds(wid * CPT, CPT)], dst_v)
    zero16 = jnp.zeros((16,), jnp.float32)

    def _fz(r, _):
        for k in range(HD // 16):
            zb[r, pl.ds(k * 16, 16)] = zero16
        return 0

    lax.fori_loop(0, 128, _fz, 0)

    def _zc(t, _):
        pltpu.sync_copy(zb, acc_sh.at[pl.ds(sid * RPS + t * 128, 128)])
        return 0

    lax.fori_loop(0, RPS // 128, _zc, 0)
    plsc.subcore_barrier()

    def _edge(j, _):
        pltpu.sync_copy(hs_hbm.at[src_v.at[j]], rows_v)
        pltpu.sync_copy(rows_v, acc_sh.at[dst_v.at[j]], add=True)
        return 0

    lax.fori_loop(0, CPT, _edge, 0)
    plsc.subcore_barrier()

    def _wb(t, _):
        pltpu.sync_copy(acc_sh.at[pl.ds(sid * RPS + t * 128, 128)], zb)
        pltpu.sync_copy(zb, out_hbm.at[cid, pl.ds(sid * RPS + t * 128, 128)])
        return 0

    lax.fori_loop(0, RPS // 128, _wb, 0)


@functools.partial(
    pl.kernel,
    mesh=_mesh,
    out_type=jax.ShapeDtypeStruct((NTILES, G, D), jnp.float32),
    compiler_params=_sc_params,
    scratch_types=[
        pltpu.VMEM((RPT, HD), jnp.float32),
        pltpu.VMEM((RPT, HD), jnp.float32),
        pltpu.VMEM((136, D), jnp.float32),
        pltpu.VMEM((2, D), jnp.float32),
        pltpu.VMEM((RPT,), jnp.int32),
    ],
)
def _sc_pool(clo_hbm, chi_hbm, batch_hbm, ac_hbm, out_hbm, cl, ch, pv, acv, bv):
    cid = lax.axis_index("c")
    sid = lax.axis_index("s")
    wid = sid * 2 + cid
    r0 = wid * RPT
    pltpu.sync_copy(clo_hbm.at[pl.ds(r0, RPT)], cl)
    pltpu.sync_copy(chi_hbm.at[pl.ds(r0, RPT)], ch)
    pltpu.sync_copy(batch_hbm.at[pl.ds(r0, RPT)], bv)
    pltpu.sync_copy(ac_hbm, acv)
    neg16 = jnp.full((16,), -1e30, jnp.float32)

    def _init(r, _):
        for k in range(8):
            pv[r, pl.ds(k * 16, 16)] = neg16
        return 0

    lax.fori_loop(0, 136, _init, 0)

    def _grp(t, _):
        gvec = bv[pl.ds(t * 16, 16)]
        for l in range(16):
            i = t * 16 + l
            g = gvec[l]
            for k in range(8):
                a = acv[0, pl.ds(k * 16, 16)]
                c = acv[1, pl.ds(k * 16, 16)]
                src = cl if k < 4 else ch
                v = src[i, pl.ds((k % 4) * 16, 16)]
                h = jnp.maximum(v * a + c, 0.0)
                cur = pv[g, pl.ds(k * 16, 16)]
                pv[g, pl.ds(k * 16, 16)] = jnp.maximum(cur, h)
        return 0

    lax.fori_loop(0, RPT // 16, _grp, 0)
    pltpu.sync_copy(pv.at[pl.ds(0, G)], out_hbm.at[wid])


def _mm_body(x_ref, wl_ref, wh_ref, deg_ref, lo_ref, hi_ref, dinv_ref):
    i = pl.program_id(0)
    deg = deg_ref[0] + deg_ref[1] + 1.0
    dinv = lax.rsqrt(deg)
    rows = lax.broadcasted_iota(jnp.int32, (1024, 1), 0) + i * 1024
    dinv = jnp.where(rows < N, dinv, 0.0)
    dinv_ref[...] = dinv
    x = x_ref[...]
    lo_ref[...] = lax.dot_general(
        x, wl_ref[...], (((1,), (0,)), ((), ())),
        precision=lax.Precision.HIGHEST,
        preferred_element_type=jnp.float32) * dinv
    hi_ref[...] = lax.dot_general(
        x, wh_ref[...], (((1,), (0,)), ((), ())),
        precision=lax.Precision.HIGHEST,
        preferred_element_type=jnp.float32) * dinv


_tc_mm_scale = pl.pallas_call(
    _mm_body,
    grid=(10,),
    in_specs=[pl.BlockSpec((1024, D), lambda i: (i, 0)),
              pl.BlockSpec((D, HD), lambda i: (0, 0)),
              pl.BlockSpec((D, HD), lambda i: (0, 0)),
              pl.BlockSpec((2, 1024, 1), lambda i: (0, i, 0))],
    out_specs=[pl.BlockSpec((1024, HD), lambda i: (i, 0)),
               pl.BlockSpec((1024, HD), lambda i: (i, 0)),
               pl.BlockSpec((1024, 1), lambda i: (i, 0))],
    out_shape=[jax.ShapeDtypeStruct((NP, HD), jnp.float32),
               jax.ShapeDtypeStruct((NP, HD), jnp.float32),
               jax.ShapeDtypeStruct((NP, 1), jnp.float32)],
)


def _stats_body(p_ref, hs_ref, dinv_ref, b_ref, g_ref, be_ref,
                conv_ref, a_ref, c_ref, s_ref, ss_ref):
    i = pl.program_id(0)
    conv = dinv_ref[...] * (p_ref[0] + p_ref[1] + hs_ref[...]) + b_ref[...]
    rows = lax.broadcasted_iota(jnp.int32, (1024, HD), 0) + i * 1024
    conv = jnp.where(rows < N, conv, 0.0)
    conv_ref[...] = conv
    ps = jnp.sum(conv, axis=0, keepdims=True)
    pss = jnp.sum(conv * conv, axis=0, keepdims=True)

    @pl.when(i == 0)
    def _():
        s_ref[...] = ps
        ss_ref[...] = pss

    @pl.when(i > 0)
    def _():
        s_ref[...] += ps
        ss_ref[...] += pss

    @pl.when(i == 9)
    def _():
        mean = s_ref[...] * (1.0 / N)
        var = ss_ref[...] * (1.0 / N) - mean * mean
        a = g_ref[...] * lax.rsqrt(var + EPS)
        a_ref[...] = a
        c_ref[...] = be_ref[...] - mean * a


_tc_stats = pl.pallas_call(
    _stats_body,
    grid=(10,),
    in_specs=[pl.BlockSpec((2, 1024, HD), lambda i: (0, i, 0)),
              pl.BlockSpec((1024, HD), lambda i: (i, 0)),
              pl.BlockSpec((1024, 1), lambda i: (i, 0)),
              pl.BlockSpec((1, HD), lambda i: (0, 0)),
              pl.BlockSpec((1, HD), lambda i: (0, 0)),
              pl.BlockSpec((1, HD), lambda i: (0, 0))],
    out_specs=[pl.BlockSpec((1024, HD), lambda i: (i, 0)),
               pl.BlockSpec((1, HD), lambda i: (0, 0)),
               pl.BlockSpec((1, HD), lambda i: (0, 0))],
    out_shape=[jax.ShapeDtypeStruct((NP, HD), jnp.float32),
               jax.ShapeDtypeStruct((1, HD), jnp.float32),
               jax.ShapeDtypeStruct((1, HD), jnp.float32)],
    scratch_shapes=[pltpu.VMEM((1, HD), jnp.float32),
                    pltpu.VMEM((1, HD), jnp.float32)],
)


def _next_body(cl_ref, ch_ref, al_ref, cll_ref, ah_ref, chh_ref,
               q00_ref, q01_ref, q10_ref, q11_ref, dinv_ref,
               lo_ref, hi_ref):
    hl = jnp.maximum(cl_ref[...] * al_ref[...] + cll_ref[...], 0.0)
    hh = jnp.maximum(ch_ref[...] * ah_ref[...] + chh_ref[...], 0.0)

    def mm(a, b):
        return lax.dot_general(a, b, (((1,), (0,)), ((), ())),
                               precision=lax.Precision.HIGHEST,
                               preferred_element_type=jnp.float32)

    dinv = dinv_ref[...]
    lo_ref[...] = (mm(hl, q00_ref[...]) + mm(hh, q10_ref[...])) * dinv
    hi_ref[...] = (mm(hl, q01_ref[...]) + mm(hh, q11_ref[...])) * dinv


_tc_next = pl.pallas_call(
    _next_body,
    grid=(10,),
    in_specs=[pl.BlockSpec((1024, HD), lambda i: (i, 0)),
              pl.BlockSpec((1024, HD), lambda i: (i, 0)),
              pl.BlockSpec((1, HD), lambda i: (0, 0)),
              pl.BlockSpec((1, HD), lambda i: (0, 0)),
              pl.BlockSpec((1, HD), lambda i: (0, 0)),
              pl.BlockSpec((1, HD), lambda i: (0, 0)),
              pl.BlockSpec((HD, HD), lambda i: (0, 0)),
              pl.BlockSpec((HD, HD), lambda i: (0, 0)),
              pl.BlockSpec((HD, HD), lambda i: (0, 0)),
              pl.BlockSpec((HD, HD), lambda i: (0, 0)),
              pl.BlockSpec((1024, 1), lambda i: (i, 0))],
    out_specs=[pl.BlockSpec((1024, HD), lambda i: (i, 0)),
               pl.BlockSpec((1024, HD), lambda i: (i, 0))],
    out_shape=[jax.ShapeDtypeStruct((NP, HD), jnp.float32),
               jax.ShapeDtypeStruct((NP, HD), jnp.float32)],
)


def _head_body(pool_ref, w1_ref, b1_ref, g_ref, be_ref, w2_ref, b2_ref, o_ref):
    pooled = jnp.max(pool_ref[...], axis=0)
    z = lax.dot_general(pooled, w1_ref[...], (((1,), (0,)), ((), ())),
                        precision=lax.Precision.HIGHEST,
                        preferred_element_type=jnp.float32) + b1_ref[...]
    mean = jnp.mean(z, axis=0, keepdims=True)
    var = jnp.mean(z * z, axis=0, keepdims=True) - mean * mean
    zn = (z - mean) * lax.rsqrt(var + EPS) * g_ref[...] + be_ref[...]
    zr = jnp.maximum(zn, 0.0)
    o = lax.dot_general(zr, w2_ref[...], (((1,), (0,)), ((), ())),
                        precision=lax.Precision.HIGHEST,
                        preferred_element_type=jnp.float32) + b2_ref[...]
    m = jnp.max(o, axis=1, keepdims=True)
    lse = jnp.log(jnp.sum(jnp.exp(o - m), axis=1, keepdims=True)) + m
    o_ref[...] = o - lse


_tc_head = pl.pallas_call(
    _head_body,
    out_shape=jax.ShapeDtypeStruct((G, 16), jnp.float32),
)


def _layer(hs_lo, hs_hi, src2d, dst2d, dinv, b, gg, be):
    p_lo = _sc_mp(hs_lo, src2d, dst2d)
    p_hi = _sc_mp(hs_hi, src2d, dst2d)
    clo, alo, cclo = _tc_stats(p_lo, hs_lo, dinv, b[:, :HD], gg[:, :HD],
                               be[:, :HD])
    chi, ahi, cchi = _tc_stats(p_hi, hs_hi, dinv, b[:, HD:], gg[:, HD:],
                               be[:, HD:])
    return clo, chi, alo, cclo, ahi, cchi


def kernel(x, edge_index, batch, W1, b1, g1, be1, W2, b2, g2, be2,
           Wl1, bl1, g3, be3, Wl2, bl2):
    src = edge_index[0].astype(jnp.int32)
    dst = edge_index[1].astype(jnp.int32)
    padE = EP - E
    src2d = jnp.concatenate(
        [src, jnp.full((padE,), DUMMY, jnp.int32)]).reshape(NTILES * CPT, CH)
    dst2d = jnp.concatenate(
        [dst, jnp.full((padE,), DUMMY, jnp.int32)]).reshape(NTILES * CPT, CH)
    batch_p = jnp.concatenate(
        [batch.astype(jnp.int32), jnp.full((NP - N,), G, jnp.int32)])
    x_p = jnp.concatenate([x, jnp.zeros((NP - N, D), x.dtype)])

    deg = _sc_degree(dst2d)
    hs1_lo, hs1_hi, dinv = _tc_mm_scale(x_p, W1[:, :HD], W1[:, HD:],
                                        deg.reshape(2, NP, 1))
    c1lo, c1hi, a1l, c1l, a1h, c1h = _layer(
        hs1_lo, hs1_hi, src2d, dst2d, dinv,
        b1.reshape(1, D), g1.reshape(1, D), be1.reshape(1, D))
    hs2_lo, hs2_hi = _tc_next(c1lo, c1hi, a1l, c1l, a1h, c1h,
                              W2[:HD, :HD], W2[:HD, HD:],
                              W2[HD:, :HD], W2[HD:, HD:], dinv)
    c2lo, c2hi, a2l, c2l, a2h, c2h = _layer(
        hs2_lo, hs2_hi, src2d, dst2d, dinv,
        b2.reshape(1, D), g2.reshape(1, D), be2.reshape(1, D))
    ac = jnp.concatenate(
        [jnp.concatenate([a2l, a2h], axis=1),
         jnp.concatenate([c2l, c2h], axis=1)], axis=0)
    pool = _sc_pool(c2lo, c2hi, batch_p, ac)
    return _tc_head(pool, Wl1, bl1.reshape(1, D), g3.reshape(1, D),
                    be3.reshape(1, D), Wl2, bl2.reshape(1, 16))


# speedup vs baseline: 7.8334x; 1.1506x over previous
import functools

import jax
import jax.numpy as jnp
from jax import lax
from jax.experimental import pallas as pl
from jax.experimental.pallas import tpu as pltpu
from jax.experimental.pallas import tpu_sc as plsc

N = 10000
NP = 10240
E = 320000
G = 128
D = 128
HD = 64
EPS = 1e-5
NTILES = 32
CH = 128
CPT = 80
EP = NTILES * CPT * CH
DUMMY = NP - 1
RPS = NP // 16
RPT = NP // NTILES
NB = 4

_mesh = plsc.VectorSubcoreMesh(core_axis_name="c", subcore_axis_name="s")
_sc_params = pltpu.CompilerParams(use_tc_tiling_on_sc=False)


@functools.partial(
    pl.kernel,
    mesh=_mesh,
    out_type=jax.ShapeDtypeStruct((2, NP), jnp.float32),
    scratch_types=[
        pltpu.VMEM((CPT, CH), jnp.int32),
        pltpu.VMEM((CH,), jnp.float32),
        pltpu.VMEM((RPS,), jnp.float32),
        pltpu.VMEM_SHARED((NP,), jnp.float32),
    ],
)
def _sc_degree(dst_hbm, out_hbm, dst_v, ones_v, stage_v, acc_sh):
    cid = lax.axis_index("c")
    sid = lax.axis_index("s")
    wid = sid * 2 + cid
    pltpu.sync_copy(dst_hbm.at[pl.ds(wid * CPT, CPT)], dst_v)
    ones16 = jnp.ones((16,), jnp.float32)
    zero16 = jnp.zeros((16,), jnp.float32)

    def _fill(i, _):
        ones_v[pl.ds(i * 16, 16)] = ones16
        return 0

    lax.fori_loop(0, CH // 16, _fill, 0)

    def _fillz(i, _):
        stage_v[pl.ds(i * 16, 16)] = zero16
        return 0

    lax.fori_loop(0, RPS // 16, _fillz, 0)
    pltpu.sync_copy(stage_v, acc_sh.at[pl.ds(sid * RPS, RPS)])
    plsc.subcore_barrier()

    def _scat(j, _):
        pltpu.sync_copy(ones_v, acc_sh.at[dst_v.at[j]], add=True)
        return 0

    lax.fori_loop(0, CPT, _scat, 0)
    plsc.subcore_barrier()
    pltpu.sync_copy(acc_sh.at[pl.ds(sid * RPS, RPS)], stage_v)
    pltpu.sync_copy(stage_v, out_hbm.at[cid, pl.ds(sid * RPS, RPS)])


@functools.partial(
    pl.kernel,
    mesh=_mesh,
    out_type=jax.ShapeDtypeStruct((2, NP, HD), jnp.float32),
    compiler_params=_sc_params,
    scratch_types=[
        pltpu.VMEM((CPT, CH), jnp.int32),
        pltpu.VMEM((CPT, CH), jnp.int32),
        [pltpu.VMEM((CH, HD), jnp.float32) for _ in range(NB)],
        pltpu.VMEM((128, HD), jnp.float32),
        pltpu.VMEM_SHARED((NP, HD), jnp.float32),
        [pltpu.SemaphoreType.DMA for _ in range(NB)],
        [pltpu.SemaphoreType.DMA for _ in range(NB)],
    ],
)
def _sc_mp(hs_hbm, src_hbm, dst_hbm, out_hbm, src_v, dst_v, bufs, zb, acc_sh,
           gsem, ssem):
    cid = lax.axis_index("c")
    sid = lax.axis_index("s")
    wid = sid * 2 + cid
    pltpu.sync_copy(src_hbm.at[pl.ds(wid * CPT, CPT)], src_v)
    pltpu.sync_copy(dst_hbm.at[pl.ds(wid * CPT, CPT)], dst_v)
    zero16 = jnp.zeros((16,), jnp.float32)

    def _fz(r, _):
        for k in range(HD // 16):
            zb[r, pl.ds(k * 16, 16)] = zero16
        return 0

    lax.fori_loop(0, 128, _fz, 0)

    def _zc(t, _):
        pltpu.sync_copy(zb, acc_sh.at[pl.ds(sid * RPS + t * 128, 128)])
        return 0

    lax.fori_loop(0, RPS // 128, _zc, 0)
    plsc.subcore_barrier()

    def _start_gather(b, j):
        pltpu.async_copy(hs_hbm.at[src_v.at[j]], bufs[b], gsem[b])

    def _wait_gather(b):
        pltpu.make_async_copy(hs_hbm.at[src_v.at[0]], bufs[b], gsem[b]).wait()

    def _start_scatter(b, j):
        pltpu.async_copy(bufs[b], acc_sh.at[dst_v.at[j]], ssem[b], add=True)

    def _wait_scatter(b):
        pltpu.make_async_copy(bufs[b], acc_sh.at[pl.ds(0, CH)], ssem[b]).wait()

    for b in range(NB):
        _start_gather(b, b)

    def _ring(t, _):
        for b in range(NB):
            _wait_gather(b)
            _start_scatter(b, t * NB + b)
        for b in range(NB):
            @pl.when(t < CPT // NB - 1)
            def _():
                _wait_scatter(b)
                _start_gather(b, (t + 1) * NB + b)
        return 0

    lax.fori_loop(0, CPT // NB, _ring, 0)
    for b in range(NB):
        _wait_scatter(b)
    plsc.subcore_barrier()

    def _wb(t, _):
        pltpu.sync_copy(acc_sh.at[pl.ds(sid * RPS + t * 128, 128)], zb)
        pltpu.sync_copy(zb, out_hbm.at[cid, pl.ds(sid * RPS + t * 128, 128)])
        return 0

    lax.fori_loop(0, RPS // 128, _wb, 0)


@functools.partial(
    pl.kernel,
    mesh=_mesh,
    out_type=jax.ShapeDtypeStruct((NTILES, G, D), jnp.float32),
    compiler_params=_sc_params,
    scratch_types=[
        pltpu.VMEM((RPT, HD), jnp.float32),
        pltpu.VMEM((RPT, HD), jnp.float32),
        pltpu.VMEM((136, D), jnp.float32),
        pltpu.VMEM((2, D), jnp.float32),
        pltpu.VMEM((RPT,), jnp.int32),
    ],
)
def _sc_pool(clo_hbm, chi_hbm, batch_hbm, ac_hbm, out_hbm, cl, ch, pv, acv, bv):
    cid = lax.axis_index("c")
    sid = lax.axis_index("s")
    wid = sid * 2 + cid
    r0 = wid * RPT
    pltpu.sync_copy(clo_hbm.at[pl.ds(r0, RPT)], cl)
    pltpu.sync_copy(chi_hbm.at[pl.ds(r0, RPT)], ch)
    pltpu.sync_copy(batch_hbm.at[pl.ds(r0, RPT)], bv)
    pltpu.sync_copy(ac_hbm, acv)
    neg16 = jnp.full((16,), -1e30, jnp.float32)

    def _init(r, _):
        for k in range(8):
            pv[r, pl.ds(k * 16, 16)] = neg16
        return 0

    lax.fori_loop(0, 136, _init, 0)

    def _grp(t, _):
        gvec = bv[pl.ds(t * 16, 16)]
        for l in range(16):
            i = t * 16 + l
            g = gvec[l]
            for k in range(8):
                a = acv[0, pl.ds(k * 16, 16)]
                c = acv[1, pl.ds(k * 16, 16)]
                src = cl if k < 4 else ch
                v = src[i, pl.ds((k % 4) * 16, 16)]
                h = jnp.maximum(v * a + c, 0.0)
                cur = pv[g, pl.ds(k * 16, 16)]
                pv[g, pl.ds(k * 16, 16)] = jnp.maximum(cur, h)
        return 0

    lax.fori_loop(0, RPT // 16, _grp, 0)
    pltpu.sync_copy(pv.at[pl.ds(0, G)], out_hbm.at[wid])


def _mm_body(x_ref, wl_ref, wh_ref, deg_ref, lo_ref, hi_ref, dinv_ref):
    i = pl.program_id(0)
    deg = deg_ref[0] + deg_ref[1] + 1.0
    dinv = lax.rsqrt(deg)
    rows = lax.broadcasted_iota(jnp.int32, (1024, 1), 0) + i * 1024
    dinv = jnp.where(rows < N, dinv, 0.0)
    dinv_ref[...] = dinv
    x = x_ref[...]
    lo_ref[...] = lax.dot_general(
        x, wl_ref[...], (((1,), (0,)), ((), ())),
        precision=lax.Precision.HIGHEST,
        preferred_element_type=jnp.float32) * dinv
    hi_ref[...] = lax.dot_general(
        x, wh_ref[...], (((1,), (0,)), ((), ())),
        precision=lax.Precision.HIGHEST,
        preferred_element_type=jnp.float32) * dinv


_tc_mm_scale = pl.pallas_call(
    _mm_body,
    grid=(10,),
    in_specs=[pl.BlockSpec((1024, D), lambda i: (i, 0)),
              pl.BlockSpec((D, HD), lambda i: (0, 0)),
              pl.BlockSpec((D, HD), lambda i: (0, 0)),
              pl.BlockSpec((2, 1024, 1), lambda i: (0, i, 0))],
    out_specs=[pl.BlockSpec((1024, HD), lambda i: (i, 0)),
               pl.BlockSpec((1024, HD), lambda i: (i, 0)),
               pl.BlockSpec((1024, 1), lambda i: (i, 0))],
    out_shape=[jax.ShapeDtypeStruct((NP, HD), jnp.float32),
               jax.ShapeDtypeStruct((NP, HD), jnp.float32),
               jax.ShapeDtypeStruct((NP, 1), jnp.float32)],
)


def _stats_body(p_ref, hs_ref, dinv_ref, b_ref, g_ref, be_ref,
                conv_ref, a_ref, c_ref, s_ref, ss_ref):
    i = pl.program_id(0)
    conv = dinv_ref[...] * (p_ref[0] + p_ref[1] + hs_ref[...]) + b_ref[...]
    rows = lax.broadcasted_iota(jnp.int32, (1024, HD), 0) + i * 1024
    conv = jnp.where(rows < N, conv, 0.0)
    conv_ref[...] = conv
    ps = jnp.sum(conv, axis=0, keepdims=True)
    pss = jnp.sum(conv * conv, axis=0, keepdims=True)

    @pl.when(i == 0)
    def _():
        s_ref[...] = ps
        ss_ref[...] = pss

    @pl.when(i > 0)
    def _():
        s_ref[...] += ps
        ss_ref[...] += pss

    @pl.when(i == 9)
    def _():
        mean = s_ref[...] * (1.0 / N)
        var = ss_ref[...] * (1.0 / N) - mean * mean
        a = g_ref[...] * lax.rsqrt(var + EPS)
        a_ref[...] = a
        c_ref[...] = be_ref[...] - mean * a


_tc_stats = pl.pallas_call(
    _stats_body,
    grid=(10,),
    in_specs=[pl.BlockSpec((2, 1024, HD), lambda i: (0, i, 0)),
              pl.BlockSpec((1024, HD), lambda i: (i, 0)),
              pl.BlockSpec((1024, 1), lambda i: (i, 0)),
              pl.BlockSpec((1, HD), lambda i: (0, 0)),
              pl.BlockSpec((1, HD), lambda i: (0, 0)),
              pl.BlockSpec((1, HD), lambda i: (0, 0))],
    out_specs=[pl.BlockSpec((1024, HD), lambda i: (i, 0)),
               pl.BlockSpec((1, HD), lambda i: (0, 0)),
               pl.BlockSpec((1, HD), lambda i: (0, 0))],
    out_shape=[jax.ShapeDtypeStruct((NP, HD), jnp.float32),
               jax.ShapeDtypeStruct((1, HD), jnp.float32),
               jax.ShapeDtypeStruct((1, HD), jnp.float32)],
    scratch_shapes=[pltpu.VMEM((1, HD), jnp.float32),
                    pltpu.VMEM((1, HD), jnp.float32)],
)


def _next_body(cl_ref, ch_ref, al_ref, cll_ref, ah_ref, chh_ref,
               q00_ref, q01_ref, q10_ref, q11_ref, dinv_ref,
               lo_ref, hi_ref):
    hl = jnp.maximum(cl_ref[...] * al_ref[...] + cll_ref[...], 0.0)
    hh = jnp.maximum(ch_ref[...] * ah_ref[...] + chh_ref[...], 0.0)

    def mm(a, b):
        return lax.dot_general(a, b, (((1,), (0,)), ((), ())),
                               precision=lax.Precision.HIGHEST,
                               preferred_element_type=jnp.float32)

    dinv = dinv_ref[...]
    lo_ref[...] = (mm(hl, q00_ref[...]) + mm(hh, q10_ref[...])) * dinv
    hi_ref[...] = (mm(hl, q01_ref[...]) + mm(hh, q11_ref[...])) * dinv


_tc_next = pl.pallas_call(
    _next_body,
    grid=(10,),
    in_specs=[pl.BlockSpec((1024, HD), lambda i: (i, 0)),
              pl.BlockSpec((1024, HD), lambda i: (i, 0)),
              pl.BlockSpec((1, HD), lambda i: (0, 0)),
              pl.BlockSpec((1, HD), lambda i: (0, 0)),
              pl.BlockSpec((1, HD), lambda i: (0, 0)),
              pl.BlockSpec((1, HD), lambda i: (0, 0)),
              pl.BlockSpec((HD, HD), lambda i: (0, 0)),
              pl.BlockSpec((HD, HD), lambda i: (0, 0)),
              pl.BlockSpec((HD, HD), lambda i: (0, 0)),
              pl.BlockSpec((HD, HD), lambda i: (0, 0)),
              pl.BlockSpec((1024, 1), lambda i: (i, 0))],
    out_specs=[pl.BlockSpec((1024, HD), lambda i: (i, 0)),
               pl.BlockSpec((1024, HD), lambda i: (i, 0))],
    out_shape=[jax.ShapeDtypeStruct((NP, HD), jnp.float32),
               jax.ShapeDtypeStruct((NP, HD), jnp.float32)],
)


def _head_body(pool_ref, w1_ref, b1_ref, g_ref, be_ref, w2_ref, b2_ref, o_ref):
    pooled = jnp.max(pool_ref[...], axis=0)
    z = lax.dot_general(pooled, w1_ref[...], (((1,), (0,)), ((), ())),
                        precision=lax.Precision.HIGHEST,
                        preferred_element_type=jnp.float32) + b1_ref[...]
    mean = jnp.mean(z, axis=0, keepdims=True)
    var = jnp.mean(z * z, axis=0, keepdims=True) - mean * mean
    zn = (z - mean) * lax.rsqrt(var + EPS) * g_ref[...] + be_ref[...]
    zr = jnp.maximum(zn, 0.0)
    o = lax.dot_general(zr, w2_ref[...], (((1,), (0,)), ((), ())),
                        precision=lax.Precision.HIGHEST,
                        preferred_element_type=jnp.float32) + b2_ref[...]
    m = jnp.max(o, axis=1, keepdims=True)
    lse = jnp.log(jnp.sum(jnp.exp(o - m), axis=1, keepdims=True)) + m
    o_ref[...] = o - lse


_tc_head = pl.pallas_call(
    _head_body,
    out_shape=jax.ShapeDtypeStruct((G, 16), jnp.float32),
)


def _layer(hs_lo, hs_hi, src2d, dst2d, dinv, b, gg, be):
    p_lo = _sc_mp(hs_lo, src2d, dst2d)
    tok = (p_lo[0, 0, 0] * 0.0).astype(jnp.int32)
    p_hi = _sc_mp(hs_hi, src2d + tok, dst2d)
    clo, alo, cclo = _tc_stats(p_lo, hs_lo, dinv, b[:, :HD], gg[:, :HD],
                               be[:, :HD])
    chi, ahi, cchi = _tc_stats(p_hi, hs_hi, dinv, b[:, HD:], gg[:, HD:],
                               be[:, HD:])
    return clo, chi, alo, cclo, ahi, cchi


def kernel(x, edge_index, batch, W1, b1, g1, be1, W2, b2, g2, be2,
           Wl1, bl1, g3, be3, Wl2, bl2):
    src = edge_index[0].astype(jnp.int32)
    dst = edge_index[1].astype(jnp.int32)
    padE = EP - E
    src2d = jnp.concatenate(
        [src, jnp.full((padE,), DUMMY, jnp.int32)]).reshape(NTILES * CPT, CH)
    dst2d = jnp.concatenate(
        [dst, jnp.full((padE,), DUMMY, jnp.int32)]).reshape(NTILES * CPT, CH)
    batch_p = jnp.concatenate(
        [batch.astype(jnp.int32), jnp.full((NP - N,), G, jnp.int32)])
    x_p = jnp.concatenate([x, jnp.zeros((NP - N, D), x.dtype)])

    deg = _sc_degree(dst2d)
    hs1_lo, hs1_hi, dinv = _tc_mm_scale(x_p, W1[:, :HD], W1[:, HD:],
                                        deg.reshape(2, NP, 1))
    c1lo, c1hi, a1l, c1l, a1h, c1h = _layer(
        hs1_lo, hs1_hi, src2d, dst2d, dinv,
        b1.reshape(1, D), g1.reshape(1, D), be1.reshape(1, D))
    hs2_lo, hs2_hi = _tc_next(c1lo, c1hi, a1l, c1l, a1h, c1h,
                              W2[:HD, :HD], W2[:HD, HD:],
                              W2[HD:, :HD], W2[HD:, HD:], dinv)
    c2lo, c2hi, a2l, c2l, a2h, c2h = _layer(
        hs2_lo, hs2_hi, src2d, dst2d, dinv,
        b2.reshape(1, D), g2.reshape(1, D), be2.reshape(1, D))
    ac = jnp.concatenate(
        [jnp.concatenate([a2l, a2h], axis=1),
         jnp.concatenate([c2l, c2h], axis=1)], axis=0)
    pool = _sc_pool(c2lo, c2hi, batch_p, ac)
    return _tc_head(pool, Wl1, bl1.reshape(1, D), g3.reshape(1, D),
                    be3.reshape(1, D), Wl2, bl2.reshape(1, 16))


# speedup vs baseline: 22.1186x; 2.8236x over previous
import functools

import jax
import jax.numpy as jnp
from jax import lax
from jax.experimental import pallas as pl
from jax.experimental.pallas import tpu as pltpu
from jax.experimental.pallas import tpu_sc as plsc

N = 10000
NP = 10240
E = 320000
G = 128
D = 128
HD = 64
EPS = 1e-5
NTILES = 32
CH = 128
CPT = 80
EP = NTILES * CPT * CH
DUMMY = NP - 1
RPS = NP // 16
RPT = NP // NTILES
NB = 4

_mesh = plsc.VectorSubcoreMesh(core_axis_name="c", subcore_axis_name="s")
_sc_params = pltpu.CompilerParams(use_tc_tiling_on_sc=False)


@functools.partial(
    pl.kernel,
    mesh=_mesh,
    out_type=jax.ShapeDtypeStruct((2, NP), jnp.float32),
    scratch_types=[
        pltpu.VMEM((CPT, CH), jnp.int32),
        pltpu.VMEM((CH,), jnp.float32),
        pltpu.VMEM((RPS,), jnp.float32),
        pltpu.VMEM_SHARED((NP,), jnp.float32),
    ],
)
def _sc_degree(dst_hbm, out_hbm, dst_v, ones_v, stage_v, acc_sh):
    cid = lax.axis_index("c")
    sid = lax.axis_index("s")
    wid = sid * 2 + cid
    pltpu.sync_copy(dst_hbm.at[pl.ds(wid * CPT, CPT)], dst_v)
    ones16 = jnp.ones((16,), jnp.float32)
    zero16 = jnp.zeros((16,), jnp.float32)

    def _fill(i, _):
        ones_v[pl.ds(i * 16, 16)] = ones16
        return 0

    lax.fori_loop(0, CH // 16, _fill, 0)

    def _fillz(i, _):
        stage_v[pl.ds(i * 16, 16)] = zero16
        return 0

    lax.fori_loop(0, RPS // 16, _fillz, 0)
    pltpu.sync_copy(stage_v, acc_sh.at[pl.ds(sid * RPS, RPS)])
    plsc.subcore_barrier()

    def _scat(j, _):
        pltpu.sync_copy(ones_v, acc_sh.at[dst_v.at[j]], add=True)
        return 0

    lax.fori_loop(0, CPT, _scat, 0)
    plsc.subcore_barrier()
    pltpu.sync_copy(acc_sh.at[pl.ds(sid * RPS, RPS)], stage_v)
    pltpu.sync_copy(stage_v, out_hbm.at[cid, pl.ds(sid * RPS, RPS)])


@functools.partial(
    pl.kernel,
    mesh=_mesh,
    out_type=jax.ShapeDtypeStruct((2, NP, HD), jnp.float32),
    compiler_params=_sc_params,
    scratch_types=[
        pltpu.VMEM((CPT, CH), jnp.int32),
        pltpu.VMEM((CPT, CH), jnp.int32),
        [pltpu.VMEM((CH, HD), jnp.float32) for _ in range(NB)],
        pltpu.VMEM((128, HD), jnp.float32),
        pltpu.VMEM_SHARED((NP, HD), jnp.float32),
        [pltpu.SemaphoreType.DMA for _ in range(NB)],
        [pltpu.SemaphoreType.DMA for _ in range(NB)],
    ],
)
def _sc_mp(hs_hbm, src_hbm, dst_hbm, out_hbm, src_v, dst_v, bufs, zb, acc_sh,
           gsem, ssem):
    cid = lax.axis_index("c")
    sid = lax.axis_index("s")
    wid = sid * 2 + cid
    pltpu.sync_copy(src_hbm.at[pl.ds(wid * CPT, CPT)], src_v)
    pltpu.sync_copy(dst_hbm.at[pl.ds(wid * CPT, CPT)], dst_v)
    zero16 = jnp.zeros((16,), jnp.float32)

    def _fz(r, _):
        for k in range(HD // 16):
            zb[r, pl.ds(k * 16, 16)] = zero16
        return 0

    lax.fori_loop(0, 128, _fz, 0)

    def _zc(t, _):
        pltpu.sync_copy(zb, acc_sh.at[pl.ds(sid * RPS + t * 128, 128)])
        return 0

    lax.fori_loop(0, RPS // 128, _zc, 0)
    plsc.subcore_barrier()

    def _start_gather(b, j):
        pltpu.async_copy(hs_hbm.at[src_v.at[j]], bufs[b], gsem[b])

    def _wait_gather(b):
        pltpu.make_async_copy(hs_hbm.at[src_v.at[0]], bufs[b], gsem[b]).wait()

    def _start_scatter(b, j):
        pltpu.async_copy(bufs[b], acc_sh.at[dst_v.at[j]], ssem[b], add=True)

    def _wait_scatter(b):
        pltpu.make_async_copy(bufs[b], acc_sh.at[pl.ds(0, CH)], ssem[b]).wait()

    for b in range(NB):
        _start_gather(b, b)

    def _ring(t, _):
        for b in range(NB):
            _wait_gather(b)
            _start_scatter(b, t * NB + b)
        for b in range(NB):
            @pl.when(t < CPT // NB - 1)
            def _():
                _wait_scatter(b)
                _start_gather(b, (t + 1) * NB + b)
        return 0

    lax.fori_loop(0, CPT // NB, _ring, 0)
    for b in range(NB):
        _wait_scatter(b)
    plsc.subcore_barrier()

    def _wb(t, _):
        pltpu.sync_copy(acc_sh.at[pl.ds(sid * RPS + t * 128, 128)], zb)
        pltpu.sync_copy(zb, out_hbm.at[cid, pl.ds(sid * RPS + t * 128, 128)])
        return 0

    lax.fori_loop(0, RPS // 128, _wb, 0)


@functools.partial(
    pl.kernel,
    mesh=_mesh,
    out_type=jax.ShapeDtypeStruct((NTILES, G, D), jnp.float32),
    compiler_params=_sc_params,
    scratch_types=[
        pltpu.VMEM((RPT, HD), jnp.float32),
        pltpu.VMEM((RPT, HD), jnp.float32),
        pltpu.VMEM((136, D), jnp.float32),
        pltpu.VMEM((2, D), jnp.float32),
        pltpu.VMEM((RPT,), jnp.int32),
    ],
)
def _sc_pool(clo_hbm, chi_hbm, batch_hbm, ac_hbm, out_hbm, cl, ch, pv, acv, bv):
    cid = lax.axis_index("c")
    sid = lax.axis_index("s")
    wid = sid * 2 + cid
    r0 = wid * RPT
    pltpu.sync_copy(clo_hbm.at[pl.ds(r0, RPT)], cl)
    pltpu.sync_copy(chi_hbm.at[pl.ds(r0, RPT)], ch)
    pltpu.sync_copy(batch_hbm.at[pl.ds(r0, RPT)], bv)
    pltpu.sync_copy(ac_hbm, acv)
    neg16 = jnp.full((16,), -1e30, jnp.float32)

    def _init(r, _):
        for k in range(8):
            pv[r, pl.ds(k * 16, 16)] = neg16
        return 0

    lax.fori_loop(0, 136, _init, 0)

    def _grp(t, _):
        gvec = bv[pl.ds(t * 16, 16)]
        for l in range(16):
            i = t * 16 + l
            g = gvec[l]
            for k in range(8):
                a = acv[0, pl.ds(k * 16, 16)]
                c = acv[1, pl.ds(k * 16, 16)]
                src = cl if k < 4 else ch
                v = src[i, pl.ds((k % 4) * 16, 16)]
                h = jnp.maximum(v * a + c, 0.0)
                cur = pv[g, pl.ds(k * 16, 16)]
                pv[g, pl.ds(k * 16, 16)] = jnp.maximum(cur, h)
        return 0

    lax.fori_loop(0, RPT // 16, _grp, 0)
    pltpu.sync_copy(pv.at[pl.ds(0, G)], out_hbm.at[wid])


def _mm_body(x_ref, wl_ref, wh_ref, deg_ref, lo_ref, hi_ref, dinv_ref):
    i = pl.program_id(0)
    deg = deg_ref[0] + deg_ref[1] + 1.0
    dinv = lax.rsqrt(deg)
    rows = lax.broadcasted_iota(jnp.int32, (1024, 1), 0) + i * 1024
    dinv = jnp.where(rows < N, dinv, 0.0)
    dinv_ref[...] = dinv
    x = x_ref[...]
    lo_ref[...] = lax.dot_general(
        x, wl_ref[...], (((1,), (0,)), ((), ())),
        precision=lax.Precision.HIGHEST,
        preferred_element_type=jnp.float32) * dinv
    hi_ref[...] = lax.dot_general(
        x, wh_ref[...], (((1,), (0,)), ((), ())),
        precision=lax.Precision.HIGHEST,
        preferred_element_type=jnp.float32) * dinv


_tc_mm_scale = pl.pallas_call(
    _mm_body,
    grid=(10,),
    in_specs=[pl.BlockSpec((1024, D), lambda i: (i, 0)),
              pl.BlockSpec((D, HD), lambda i: (0, 0)),
              pl.BlockSpec((D, HD), lambda i: (0, 0)),
              pl.BlockSpec((2, 1024, 1), lambda i: (0, i, 0))],
    out_specs=[pl.BlockSpec((1024, HD), lambda i: (i, 0)),
               pl.BlockSpec((1024, HD), lambda i: (i, 0)),
               pl.BlockSpec((1024, 1), lambda i: (i, 0))],
    out_shape=[jax.ShapeDtypeStruct((NP, HD), jnp.float32),
               jax.ShapeDtypeStruct((NP, HD), jnp.float32),
               jax.ShapeDtypeStruct((NP, 1), jnp.float32)],
)


def _stats_body(p_ref, hs_ref, dinv_ref, b_ref, g_ref, be_ref,
                conv_ref, a_ref, c_ref, s_ref, ss_ref):
    i = pl.program_id(0)
    conv = dinv_ref[...] * (p_ref[0] + p_ref[1] + hs_ref[...]) + b_ref[...]
    rows = lax.broadcasted_iota(jnp.int32, (1024, HD), 0) + i * 1024
    conv = jnp.where(rows < N, conv, 0.0)
    conv_ref[...] = conv
    ps = jnp.sum(conv, axis=0, keepdims=True)
    pss = jnp.sum(conv * conv, axis=0, keepdims=True)

    @pl.when(i == 0)
    def _():
        s_ref[...] = ps
        ss_ref[...] = pss

    @pl.when(i > 0)
    def _():
        s_ref[...] += ps
        ss_ref[...] += pss

    @pl.when(i == 9)
    def _():
        mean = s_ref[...] * (1.0 / N)
        var = ss_ref[...] * (1.0 / N) - mean * mean
        a = g_ref[...] * lax.rsqrt(var + EPS)
        a_ref[...] = a
        c_ref[...] = be_ref[...] - mean * a


_tc_stats = pl.pallas_call(
    _stats_body,
    grid=(10,),
    in_specs=[pl.BlockSpec((2, 1024, HD), lambda i: (0, i, 0)),
              pl.BlockSpec((1024, HD), lambda i: (i, 0)),
              pl.BlockSpec((1024, 1), lambda i: (i, 0)),
              pl.BlockSpec((1, HD), lambda i: (0, 0)),
              pl.BlockSpec((1, HD), lambda i: (0, 0)),
              pl.BlockSpec((1, HD), lambda i: (0, 0))],
    out_specs=[pl.BlockSpec((1024, HD), lambda i: (i, 0)),
               pl.BlockSpec((1, HD), lambda i: (0, 0)),
               pl.BlockSpec((1, HD), lambda i: (0, 0))],
    out_shape=[jax.ShapeDtypeStruct((NP, HD), jnp.float32),
               jax.ShapeDtypeStruct((1, HD), jnp.float32),
               jax.ShapeDtypeStruct((1, HD), jnp.float32)],
    scratch_shapes=[pltpu.VMEM((1, HD), jnp.float32),
                    pltpu.VMEM((1, HD), jnp.float32)],
)


def _next_body(cl_ref, ch_ref, al_ref, cll_ref, ah_ref, chh_ref,
               q00_ref, q01_ref, q10_ref, q11_ref, dinv_ref,
               lo_ref, hi_ref):
    hl = jnp.maximum(cl_ref[...] * al_ref[...] + cll_ref[...], 0.0)
    hh = jnp.maximum(ch_ref[...] * ah_ref[...] + chh_ref[...], 0.0)

    def mm(a, b):
        return lax.dot_general(a, b, (((1,), (0,)), ((), ())),
                               precision=lax.Precision.HIGHEST,
                               preferred_element_type=jnp.float32)

    dinv = dinv_ref[...]
    lo_ref[...] = (mm(hl, q00_ref[...]) + mm(hh, q10_ref[...])) * dinv
    hi_ref[...] = (mm(hl, q01_ref[...]) + mm(hh, q11_ref[...])) * dinv


_tc_next = pl.pallas_call(
    _next_body,
    grid=(10,),
    in_specs=[pl.BlockSpec((1024, HD), lambda i: (i, 0)),
              pl.BlockSpec((1024, HD), lambda i: (i, 0)),
              pl.BlockSpec((1, HD), lambda i: (0, 0)),
              pl.BlockSpec((1, HD), lambda i: (0, 0)),
              pl.BlockSpec((1, HD), lambda i: (0, 0)),
              pl.BlockSpec((1, HD), lambda i: (0, 0)),
              pl.BlockSpec((HD, HD), lambda i: (0, 0)),
              pl.BlockSpec((HD, HD), lambda i: (0, 0)),
              pl.BlockSpec((HD, HD), lambda i: (0, 0)),
              pl.BlockSpec((HD, HD), lambda i: (0, 0)),
              pl.BlockSpec((1024, 1), lambda i: (i, 0))],
    out_specs=[pl.BlockSpec((1024, HD), lambda i: (i, 0)),
               pl.BlockSpec((1024, HD), lambda i: (i, 0))],
    out_shape=[jax.ShapeDtypeStruct((NP, HD), jnp.float32),
               jax.ShapeDtypeStruct((NP, HD), jnp.float32)],
)


def _head_body(pool_ref, w1_ref, b1_ref, g_ref, be_ref, w2_ref, b2_ref, o_ref):
    pooled = jnp.max(pool_ref[...], axis=0)
    z = lax.dot_general(pooled, w1_ref[...], (((1,), (0,)), ((), ())),
                        precision=lax.Precision.HIGHEST,
                        preferred_element_type=jnp.float32) + b1_ref[...]
    mean = jnp.mean(z, axis=0, keepdims=True)
    var = jnp.mean(z * z, axis=0, keepdims=True) - mean * mean
    zn = (z - mean) * lax.rsqrt(var + EPS) * g_ref[...] + be_ref[...]
    zr = jnp.maximum(zn, 0.0)
    o = lax.dot_general(zr, w2_ref[...], (((1,), (0,)), ((), ())),
                        precision=lax.Precision.HIGHEST,
                        preferred_element_type=jnp.float32) + b2_ref[...]
    m = jnp.max(o, axis=1, keepdims=True)
    lse = jnp.log(jnp.sum(jnp.exp(o - m), axis=1, keepdims=True)) + m
    o_ref[...] = o - lse


_tc_head = pl.pallas_call(
    _head_body,
    out_shape=jax.ShapeDtypeStruct((G, 16), jnp.float32),
)


def _layer(hs_lo, hs_hi, src2d, dst2d, dinv, b, gg, be):
    p_lo = _sc_mp(hs_lo, src2d, dst2d)
    tok = (p_lo[0, 0, 0] * 0.0).astype(jnp.int32)
    p_hi = _sc_mp(hs_hi, src2d + tok, dst2d)
    clo, alo, cclo = _tc_stats(p_lo, hs_lo, dinv, b[:, :HD], gg[:, :HD],
                               be[:, :HD])
    chi, ahi, cchi = _tc_stats(p_hi, hs_hi, dinv, b[:, HD:], gg[:, HD:],
                               be[:, HD:])
    return clo, chi, alo, cclo, ahi, cchi


def kernel(x, edge_index, batch, W1, b1, g1, be1, W2, b2, g2, be2,
           Wl1, bl1, g3, be3, Wl2, bl2):
    src = edge_index[0].astype(jnp.int32)
    dst = edge_index[1].astype(jnp.int32)
    padE = EP - E
    padv = N + (jnp.arange(padE, dtype=jnp.int32) % (NP - N))
    src2d = jnp.concatenate([src, padv]).reshape(NTILES * CPT, CH)
    dst2d = jnp.concatenate([dst, padv]).reshape(NTILES * CPT, CH)
    batch_p = jnp.concatenate(
        [batch.astype(jnp.int32), jnp.full((NP - N,), G, jnp.int32)])
    x_p = jnp.concatenate([x, jnp.zeros((NP - N, D), x.dtype)])

    deg = _sc_degree(dst2d)
    hs1_lo, hs1_hi, dinv = _tc_mm_scale(x_p, W1[:, :HD], W1[:, HD:],
                                        deg.reshape(2, NP, 1))
    c1lo, c1hi, a1l, c1l, a1h, c1h = _layer(
        hs1_lo, hs1_hi, src2d, dst2d, dinv,
        b1.reshape(1, D), g1.reshape(1, D), be1.reshape(1, D))
    hs2_lo, hs2_hi = _tc_next(c1lo, c1hi, a1l, c1l, a1h, c1h,
                              W2[:HD, :HD], W2[:HD, HD:],
                              W2[HD:, :HD], W2[HD:, HD:], dinv)
    c2lo, c2hi, a2l, c2l, a2h, c2h = _layer(
        hs2_lo, hs2_hi, src2d, dst2d, dinv,
        b2.reshape(1, D), g2.reshape(1, D), be2.reshape(1, D))
    ac = jnp.concatenate(
        [jnp.concatenate([a2l, a2h], axis=1),
         jnp.concatenate([c2l, c2h], axis=1)], axis=0)
    pool = _sc_pool(c2lo, c2hi, batch_p, ac)
    return _tc_head(pool, Wl1, bl1.reshape(1, D), g3.reshape(1, D),
                    be3.reshape(1, D), Wl2, bl2.reshape(1, 16))


# speedup vs baseline: 22.7436x; 1.0283x over previous
import functools

import jax
import jax.numpy as jnp
from jax import lax
from jax.experimental import pallas as pl
from jax.experimental.pallas import tpu as pltpu
from jax.experimental.pallas import tpu_sc as plsc

N = 10000
NP = 10240
E = 320000
G = 128
D = 128
HD = 64
EPS = 1e-5
NTILES = 32
CH = 128
CPT = 80
EP = NTILES * CPT * CH
DUMMY = NP - 1
RPS = NP // 16
RPT = NP // NTILES
NB = 4

_mesh = plsc.VectorSubcoreMesh(core_axis_name="c", subcore_axis_name="s")
_sc_params = pltpu.CompilerParams(use_tc_tiling_on_sc=False)


@functools.partial(
    pl.kernel,
    mesh=_mesh,
    out_type=jax.ShapeDtypeStruct((2, NP), jnp.float32),
    scratch_types=[
        pltpu.VMEM((CPT, CH), jnp.int32),
        pltpu.VMEM((CH,), jnp.float32),
        pltpu.VMEM((RPS,), jnp.float32),
        pltpu.VMEM_SHARED((NP,), jnp.float32),
    ],
)
def _sc_degree(dst_hbm, out_hbm, dst_v, ones_v, stage_v, acc_sh):
    cid = lax.axis_index("c")
    sid = lax.axis_index("s")
    wid = sid * 2 + cid
    pltpu.sync_copy(dst_hbm.at[pl.ds(wid * CPT, CPT)], dst_v)
    ones16 = jnp.ones((16,), jnp.float32)
    zero16 = jnp.zeros((16,), jnp.float32)

    def _fill(i, _):
        ones_v[pl.ds(i * 16, 16)] = ones16
        return 0

    lax.fori_loop(0, CH // 16, _fill, 0)

    def _fillz(i, _):
        stage_v[pl.ds(i * 16, 16)] = zero16
        return 0

    lax.fori_loop(0, RPS // 16, _fillz, 0)
    pltpu.sync_copy(stage_v, acc_sh.at[pl.ds(sid * RPS, RPS)])
    plsc.subcore_barrier()

    def _scat(j, _):
        pltpu.sync_copy(ones_v, acc_sh.at[dst_v.at[j]], add=True)
        return 0

    lax.fori_loop(0, CPT, _scat, 0)
    plsc.subcore_barrier()
    pltpu.sync_copy(acc_sh.at[pl.ds(sid * RPS, RPS)], stage_v)
    pltpu.sync_copy(stage_v, out_hbm.at[cid, pl.ds(sid * RPS, RPS)])


@functools.partial(
    pl.kernel,
    mesh=_mesh,
    out_type=jax.ShapeDtypeStruct((2, NP, HD), jnp.float32),
    compiler_params=_sc_params,
    scratch_types=[
        pltpu.VMEM((CPT, CH), jnp.int32),
        pltpu.VMEM((CPT, CH), jnp.int32),
        [pltpu.VMEM((CH, HD), jnp.float32) for _ in range(NB)],
        pltpu.VMEM((128, HD), jnp.float32),
        pltpu.VMEM_SHARED((NP, HD), jnp.float32),
        [pltpu.SemaphoreType.DMA for _ in range(NB)],
        [pltpu.SemaphoreType.DMA for _ in range(NB)],
    ],
)
def _sc_mp(hs_hbm, src_hbm, dst_hbm, out_hbm, src_v, dst_v, bufs, zb, acc_sh,
           gsem, ssem):
    cid = lax.axis_index("c")
    sid = lax.axis_index("s")
    wid = sid * 2 + cid
    pltpu.async_copy(src_hbm.at[pl.ds(wid * CPT, CPT)], src_v, ssem[0])
    pltpu.async_copy(dst_hbm.at[pl.ds(wid * CPT, CPT)], dst_v, ssem[1])
    zero16 = jnp.zeros((16,), jnp.float32)

    def _fz(r, _):
        for k in range(HD // 16):
            zb[r, pl.ds(k * 16, 16)] = zero16
        return 0

    lax.fori_loop(0, 128, _fz, 0)

    for t in range(RPS // 128):
        pltpu.async_copy(zb, acc_sh.at[pl.ds(sid * RPS + t * 128, 128)],
                         gsem[0])
    for t in range(RPS // 128):
        pltpu.make_async_copy(zb, acc_sh.at[pl.ds(0, 128)], gsem[0]).wait()
    pltpu.make_async_copy(src_hbm.at[pl.ds(0, CPT)], src_v, ssem[0]).wait()
    pltpu.make_async_copy(dst_hbm.at[pl.ds(0, CPT)], dst_v, ssem[1]).wait()
    plsc.subcore_barrier()

    def _start_gather(b, j):
        pltpu.async_copy(hs_hbm.at[src_v.at[j]], bufs[b], gsem[b])

    def _wait_gather(b):
        pltpu.make_async_copy(hs_hbm.at[src_v.at[0]], bufs[b], gsem[b]).wait()

    def _start_scatter(b, j):
        pltpu.async_copy(bufs[b], acc_sh.at[dst_v.at[j]], ssem[b], add=True)

    def _wait_scatter(b):
        pltpu.make_async_copy(bufs[b], acc_sh.at[pl.ds(0, CH)], ssem[b]).wait()

    for b in range(NB):
        _start_gather(b, b)

    def _ring(t, _):
        for b in range(NB):
            _wait_gather(b)
            _start_scatter(b, t * NB + b)
        for b in range(NB):
            @pl.when(t < CPT // NB - 1)
            def _():
                _wait_scatter(b)
                _start_gather(b, (t + 1) * NB + b)
        return 0

    lax.fori_loop(0, CPT // NB, _ring, 0)
    for b in range(NB):
        _wait_scatter(b)
    plsc.subcore_barrier()

    for t in range(RPS // 128):
        b = t % 2
        if t >= 2:
            pltpu.make_async_copy(
                bufs[b], out_hbm.at[0, pl.ds(0, 128)], ssem[b]).wait()
        pltpu.async_copy(acc_sh.at[pl.ds(sid * RPS + t * 128, 128)], bufs[b],
                         gsem[b])
        pltpu.make_async_copy(
            acc_sh.at[pl.ds(0, 128)], bufs[b], gsem[b]).wait()
        pltpu.async_copy(bufs[b], out_hbm.at[cid, pl.ds(sid * RPS + t * 128,
                                                        128)], ssem[b])
    for b in range(2):
        pltpu.make_async_copy(
            bufs[b], out_hbm.at[0, pl.ds(0, 128)], ssem[b]).wait()


@functools.partial(
    pl.kernel,
    mesh=_mesh,
    out_type=jax.ShapeDtypeStruct((NTILES, G, D), jnp.float32),
    compiler_params=_sc_params,
    scratch_types=[
        pltpu.VMEM((RPT, HD), jnp.float32),
        pltpu.VMEM((RPT, HD), jnp.float32),
        pltpu.VMEM((136, D), jnp.float32),
        pltpu.VMEM((2, D), jnp.float32),
        pltpu.VMEM((RPT,), jnp.int32),
    ],
)
def _sc_pool(clo_hbm, chi_hbm, batch_hbm, ac_hbm, out_hbm, cl, ch, pv, acv, bv):
    cid = lax.axis_index("c")
    sid = lax.axis_index("s")
    wid = sid * 2 + cid
    r0 = wid * RPT
    pltpu.sync_copy(clo_hbm.at[pl.ds(r0, RPT)], cl)
    pltpu.sync_copy(chi_hbm.at[pl.ds(r0, RPT)], ch)
    pltpu.sync_copy(batch_hbm.at[pl.ds(r0, RPT)], bv)
    pltpu.sync_copy(ac_hbm, acv)
    neg16 = jnp.full((16,), -1e30, jnp.float32)

    def _init(r, _):
        for k in range(8):
            pv[r, pl.ds(k * 16, 16)] = neg16
        return 0

    lax.fori_loop(0, 136, _init, 0)

    def _grp(t, _):
        gvec = bv[pl.ds(t * 16, 16)]
        for l in range(16):
            i = t * 16 + l
            g = gvec[l]
            for k in range(8):
                a = acv[0, pl.ds(k * 16, 16)]
                c = acv[1, pl.ds(k * 16, 16)]
                src = cl if k < 4 else ch
                v = src[i, pl.ds((k % 4) * 16, 16)]
                h = jnp.maximum(v * a + c, 0.0)
                cur = pv[g, pl.ds(k * 16, 16)]
                pv[g, pl.ds(k * 16, 16)] = jnp.maximum(cur, h)
        return 0

    lax.fori_loop(0, RPT // 16, _grp, 0)
    pltpu.sync_copy(pv.at[pl.ds(0, G)], out_hbm.at[wid])


def _mm_body(x_ref, wl_ref, wh_ref, deg_ref, lo_ref, hi_ref, dinv_ref):
    i = pl.program_id(0)
    deg = deg_ref[0] + deg_ref[1] + 1.0
    dinv = lax.rsqrt(deg)
    rows = lax.broadcasted_iota(jnp.int32, (1024, 1), 0) + i * 1024
    dinv = jnp.where(rows < N, dinv, 0.0)
    dinv_ref[...] = dinv
    x = x_ref[...]
    lo_ref[...] = lax.dot_general(
        x, wl_ref[...], (((1,), (0,)), ((), ())),
        precision=lax.Precision.HIGHEST,
        preferred_element_type=jnp.float32) * dinv
    hi_ref[...] = lax.dot_general(
        x, wh_ref[...], (((1,), (0,)), ((), ())),
        precision=lax.Precision.HIGHEST,
        preferred_element_type=jnp.float32) * dinv


_tc_mm_scale = pl.pallas_call(
    _mm_body,
    grid=(10,),
    in_specs=[pl.BlockSpec((1024, D), lambda i: (i, 0)),
              pl.BlockSpec((D, HD), lambda i: (0, 0)),
              pl.BlockSpec((D, HD), lambda i: (0, 0)),
              pl.BlockSpec((2, 1024, 1), lambda i: (0, i, 0))],
    out_specs=[pl.BlockSpec((1024, HD), lambda i: (i, 0)),
               pl.BlockSpec((1024, HD), lambda i: (i, 0)),
               pl.BlockSpec((1024, 1), lambda i: (i, 0))],
    out_shape=[jax.ShapeDtypeStruct((NP, HD), jnp.float32),
               jax.ShapeDtypeStruct((NP, HD), jnp.float32),
               jax.ShapeDtypeStruct((NP, 1), jnp.float32)],
)


def _stats_body(p_ref, hs_ref, dinv_ref, b_ref, g_ref, be_ref,
                conv_ref, a_ref, c_ref, s_ref, ss_ref):
    i = pl.program_id(0)
    conv = dinv_ref[...] * (p_ref[0] + p_ref[1] + hs_ref[...]) + b_ref[...]
    rows = lax.broadcasted_iota(jnp.int32, (1024, HD), 0) + i * 1024
    conv = jnp.where(rows < N, conv, 0.0)
    conv_ref[...] = conv
    ps = jnp.sum(conv, axis=0, keepdims=True)
    pss = jnp.sum(conv * conv, axis=0, keepdims=True)

    @pl.when(i == 0)
    def _():
        s_ref[...] = ps
        ss_ref[...] = pss

    @pl.when(i > 0)
    def _():
        s_ref[...] += ps
        ss_ref[...] += pss

    @pl.when(i == 9)
    def _():
        mean = s_ref[...] * (1.0 / N)
        var = ss_ref[...] * (1.0 / N) - mean * mean
        a = g_ref[...] * lax.rsqrt(var + EPS)
        a_ref[...] = a
        c_ref[...] = be_ref[...] - mean * a


_tc_stats = pl.pallas_call(
    _stats_body,
    grid=(10,),
    in_specs=[pl.BlockSpec((2, 1024, HD), lambda i: (0, i, 0)),
              pl.BlockSpec((1024, HD), lambda i: (i, 0)),
              pl.BlockSpec((1024, 1), lambda i: (i, 0)),
              pl.BlockSpec((1, HD), lambda i: (0, 0)),
              pl.BlockSpec((1, HD), lambda i: (0, 0)),
              pl.BlockSpec((1, HD), lambda i: (0, 0))],
    out_specs=[pl.BlockSpec((1024, HD), lambda i: (i, 0)),
               pl.BlockSpec((1, HD), lambda i: (0, 0)),
               pl.BlockSpec((1, HD), lambda i: (0, 0))],
    out_shape=[jax.ShapeDtypeStruct((NP, HD), jnp.float32),
               jax.ShapeDtypeStruct((1, HD), jnp.float32),
               jax.ShapeDtypeStruct((1, HD), jnp.float32)],
    scratch_shapes=[pltpu.VMEM((1, HD), jnp.float32),
                    pltpu.VMEM((1, HD), jnp.float32)],
)


def _next_body(cl_ref, ch_ref, al_ref, cll_ref, ah_ref, chh_ref,
               q00_ref, q01_ref, q10_ref, q11_ref, dinv_ref,
               lo_ref, hi_ref):
    hl = jnp.maximum(cl_ref[...] * al_ref[...] + cll_ref[...], 0.0)
    hh = jnp.maximum(ch_ref[...] * ah_ref[...] + chh_ref[...], 0.0)

    def mm(a, b):
        return lax.dot_general(a, b, (((1,), (0,)), ((), ())),
                               precision=lax.Precision.HIGHEST,
                               preferred_element_type=jnp.float32)

    dinv = dinv_ref[...]
    lo_ref[...] = (mm(hl, q00_ref[...]) + mm(hh, q10_ref[...])) * dinv
    hi_ref[...] = (mm(hl, q01_ref[...]) + mm(hh, q11_ref[...])) * dinv


_tc_next = pl.pallas_call(
    _next_body,
    grid=(10,),
    in_specs=[pl.BlockSpec((1024, HD), lambda i: (i, 0)),
              pl.BlockSpec((1024, HD), lambda i: (i, 0)),
              pl.BlockSpec((1, HD), lambda i: (0, 0)),
              pl.BlockSpec((1, HD), lambda i: (0, 0)),
              pl.BlockSpec((1, HD), lambda i: (0, 0)),
              pl.BlockSpec((1, HD), lambda i: (0, 0)),
              pl.BlockSpec((HD, HD), lambda i: (0, 0)),
              pl.BlockSpec((HD, HD), lambda i: (0, 0)),
              pl.BlockSpec((HD, HD), lambda i: (0, 0)),
              pl.BlockSpec((HD, HD), lambda i: (0, 0)),
              pl.BlockSpec((1024, 1), lambda i: (i, 0))],
    out_specs=[pl.BlockSpec((1024, HD), lambda i: (i, 0)),
               pl.BlockSpec((1024, HD), lambda i: (i, 0))],
    out_shape=[jax.ShapeDtypeStruct((NP, HD), jnp.float32),
               jax.ShapeDtypeStruct((NP, HD), jnp.float32)],
)


def _head_body(pool_ref, w1_ref, b1_ref, g_ref, be_ref, w2_ref, b2_ref, o_ref):
    pooled = jnp.max(pool_ref[...], axis=0)
    z = lax.dot_general(pooled, w1_ref[...], (((1,), (0,)), ((), ())),
                        precision=lax.Precision.HIGHEST,
                        preferred_element_type=jnp.float32) + b1_ref[...]
    mean = jnp.mean(z, axis=0, keepdims=True)
    var = jnp.mean(z * z, axis=0, keepdims=True) - mean * mean
    zn = (z - mean) * lax.rsqrt(var + EPS) * g_ref[...] + be_ref[...]
    zr = jnp.maximum(zn, 0.0)
    o = lax.dot_general(zr, w2_ref[...], (((1,), (0,)), ((), ())),
                        precision=lax.Precision.HIGHEST,
                        preferred_element_type=jnp.float32) + b2_ref[...]
    m = jnp.max(o, axis=1, keepdims=True)
    lse = jnp.log(jnp.sum(jnp.exp(o - m), axis=1, keepdims=True)) + m
    o_ref[...] = o - lse


_tc_head = pl.pallas_call(
    _head_body,
    out_shape=jax.ShapeDtypeStruct((G, 16), jnp.float32),
)


def _layer(hs_lo, hs_hi, src2d, dst2d, dinv, b, gg, be):
    p_lo = _sc_mp(hs_lo, src2d, dst2d)
    tok = (p_lo[0, 0, 0] * 0.0).astype(jnp.int32)
    p_hi = _sc_mp(hs_hi, src2d + tok, dst2d)
    clo, alo, cclo = _tc_stats(p_lo, hs_lo, dinv, b[:, :HD], gg[:, :HD],
                               be[:, :HD])
    chi, ahi, cchi = _tc_stats(p_hi, hs_hi, dinv, b[:, HD:], gg[:, HD:],
                               be[:, HD:])
    return clo, chi, alo, cclo, ahi, cchi


def kernel(x, edge_index, batch, W1, b1, g1, be1, W2, b2, g2, be2,
           Wl1, bl1, g3, be3, Wl2, bl2):
    src = edge_index[0].astype(jnp.int32)
    dst = edge_index[1].astype(jnp.int32)
    padE = EP - E
    padv = N + (jnp.arange(padE, dtype=jnp.int32) % (NP - N))
    src2d = jnp.concatenate([src, padv]).reshape(NTILES * CPT, CH)
    dst2d = jnp.concatenate([dst, padv]).reshape(NTILES * CPT, CH)
    batch_p = jnp.concatenate(
        [batch.astype(jnp.int32), jnp.full((NP - N,), G, jnp.int32)])
    x_p = jnp.concatenate([x, jnp.zeros((NP - N, D), x.dtype)])

    deg = _sc_degree(dst2d)
    hs1_lo, hs1_hi, dinv = _tc_mm_scale(x_p, W1[:, :HD], W1[:, HD:],
                                        deg.reshape(2, NP, 1))
    c1lo, c1hi, a1l, c1l, a1h, c1h = _layer(
        hs1_lo, hs1_hi, src2d, dst2d, dinv,
        b1.reshape(1, D), g1.reshape(1, D), be1.reshape(1, D))
    hs2_lo, hs2_hi = _tc_next(c1lo, c1hi, a1l, c1l, a1h, c1h,
                              W2[:HD, :HD], W2[:HD, HD:],
                              W2[HD:, :HD], W2[HD:, HD:], dinv)
    c2lo, c2hi, a2l, c2l, a2h, c2h = _layer(
        hs2_lo, hs2_hi, src2d, dst2d, dinv,
        b2.reshape(1, D), g2.reshape(1, D), be2.reshape(1, D))
    ac = jnp.concatenate(
        [jnp.concatenate([a2l, a2h], axis=1),
         jnp.concatenate([c2l, c2h], axis=1)], axis=0)
    pool = _sc_pool(c2lo, c2hi, batch_p, ac)
    return _tc_head(pool, Wl1, bl1.reshape(1, D), g3.reshape(1, D),
                    be3.reshape(1, D), Wl2, bl2.reshape(1, 16))
